# Initial kernel scaffold; baseline (speedup 1.0000x reference)
import jax
import jax.numpy as jnp
from jax.experimental import pallas as pl


def kernel(x, edge_index, i, W1_1, W2_1, b_1, W1_2, W2_2, b_2, W1_3, W2_3, b_3, Wd, bd):
    raise NotImplementedError("write your pallas kernel here")


# speedup vs baseline: 15.9586x; 15.9586x over previous
import functools

import jax
import jax.numpy as jnp
from jax import lax
from jax.experimental import pallas as pl
from jax.experimental.pallas import tpu as pltpu
from jax.experimental.pallas import tpu_sc as plsc

N = 10000
E = 320000
F = 128
C = 32
G = 16
L = 10

NC = 2
NS = 16
NT = NC * NS
CH = 128
CPT = 80
EPT = CH * CPT
EPAD = NT * EPT
NP = 10112
RPT = NP // NS
DUMMY = N


def _deg_body(dst_hbm, ones_hbm, zeros_hbm, out_hbm, dstv, onesv, deg_sh):
    c = lax.axis_index("c")
    s = lax.axis_index("s")
    wid = s * NC + c
    pltpu.sync_copy(dst_hbm.at[wid], dstv)
    pltpu.sync_copy(ones_hbm, onesv)
    pltpu.sync_copy(zeros_hbm.at[pl.ds(s * RPT, RPT)],
                    deg_sh.at[pl.ds(s * RPT, RPT)])
    plsc.subcore_barrier()

    def body(g, carry):
        pltpu.sync_copy(onesv, deg_sh.at[dstv.at[g]], add=True)
        return carry

    lax.fori_loop(0, CPT, body, 0)
    plsc.subcore_barrier()
    pltpu.sync_copy(deg_sh.at[pl.ds(s * RPT, RPT)],
                    out_hbm.at[c, pl.ds(s * RPT, RPT)])


@functools.cache
def _deg_call():
    return pl.kernel(
        _deg_body,
        out_type=jax.ShapeDtypeStruct((NC, NP, 8), jnp.float32),
        mesh=plsc.VectorSubcoreMesh(
            core_axis_name="c", subcore_axis_name="s",
            num_cores=NC, num_subcores=NS),
        scratch_types=[
            pltpu.VMEM((CPT, CH), jnp.int32),
            pltpu.VMEM((CH, 8), jnp.float32),
            pltpu.VMEM_SHARED((NP, 8), jnp.float32),
        ],
        compiler_params=pltpu.CompilerParams(use_tc_tiling_on_sc=False),
    )


def _edge_body(hp_hbm, src_hbm, dst_hbm, zeros_hbm, out_hbm,
               srcv, dstv, rows0, rows1, agg_sh, semg0, semg1):
    c = lax.axis_index("c")
    s = lax.axis_index("s")
    wid = s * NC + c
    pltpu.sync_copy(src_hbm.at[wid], srcv)
    pltpu.sync_copy(dst_hbm.at[wid], dstv)
    pltpu.sync_copy(zeros_hbm.at[pl.ds(s * RPT, RPT)],
                    agg_sh.at[pl.ds(s * RPT, RPT)])
    plsc.subcore_barrier()

    def body(p, carry):
        g0 = 2 * p
        g1 = 2 * p + 1
        d0 = pltpu.async_copy(hp_hbm.at[srcv.at[g0]], rows0, semg0)
        d1 = pltpu.async_copy(hp_hbm.at[srcv.at[g1]], rows1, semg1)
        d0.wait()
        pltpu.sync_copy(rows0, agg_sh.at[dstv.at[g0]], add=True)
        d1.wait()
        pltpu.sync_copy(rows1, agg_sh.at[dstv.at[g1]], add=True)
        return carry

    lax.fori_loop(0, CPT // 2, body, 0)
    plsc.subcore_barrier()
    pltpu.sync_copy(agg_sh.at[pl.ds(s * RPT, RPT)],
                    out_hbm.at[c, pl.ds(s * RPT, RPT)])


@functools.cache
def _edge_call():
    return pl.kernel(
        _edge_body,
        out_type=jax.ShapeDtypeStruct((NC, NP, C), jnp.float32),
        mesh=plsc.VectorSubcoreMesh(
            core_axis_name="c", subcore_axis_name="s",
            num_cores=NC, num_subcores=NS),
        scratch_types=[
            pltpu.VMEM((CPT, CH), jnp.int32),
            pltpu.VMEM((CPT, CH), jnp.int32),
            pltpu.VMEM((CH, C), jnp.float32),
            pltpu.VMEM((CH, C), jnp.float32),
            pltpu.VMEM_SHARED((NP, C), jnp.float32),
            pltpu.SemaphoreType.DMA,
            pltpu.SemaphoreType.DMA,
        ],
        compiler_params=pltpu.CompilerParams(use_tc_tiling_on_sc=False),
    )


def _dinv_col(deg_ref):
    deg = deg_ref[0] + deg_ref[1]
    return lax.rsqrt(jnp.maximum(deg, 1.0))[:N, 0:1]


def _prep_tc(deg_ref, x_ref, w1_ref, w2_ref, b_ref, hp_ref, skip_ref):
    dcol = _dinv_col(deg_ref)
    x = x_ref[...]
    h = jnp.dot(x, w1_ref[...], preferred_element_type=jnp.float32)
    hp_ref[...] = h * dcol
    skip_ref[...] = (
        jnp.dot(x, w2_ref[...], preferred_element_type=jnp.float32) + b_ref[...]
    )


_prep_call = pl.pallas_call(
    _prep_tc,
    out_shape=(
        jax.ShapeDtypeStruct((N, C), jnp.float32),
        jax.ShapeDtypeStruct((N, C), jnp.float32),
    ),
)


def _mid_tc(agg_ref, deg_ref, skip_ref, w1_ref, w2_ref, b_ref,
            hp_ref, skip2_ref):
    dcol = _dinv_col(deg_ref)
    agg = agg_ref[0, :N] + agg_ref[1, :N]
    h = jnp.maximum(agg * dcol + skip_ref[...], 0.0)
    hp_ref[...] = (
        jnp.dot(h, w1_ref[...], preferred_element_type=jnp.float32) * dcol
    )
    skip2_ref[...] = (
        jnp.dot(h, w2_ref[...], preferred_element_type=jnp.float32) + b_ref[...]
    )


_mid_call = pl.pallas_call(
    _mid_tc,
    out_shape=(
        jax.ShapeDtypeStruct((N, C), jnp.float32),
        jax.ShapeDtypeStruct((N, C), jnp.float32),
    ),
)


def _final_tc(agg_ref, deg_ref, skip_ref, seg_ref, wd_ref, bd_ref, out_ref):
    dcol = _dinv_col(deg_ref)
    agg = agg_ref[0, :N] + agg_ref[1, :N]
    h = jnp.maximum(agg * dcol + skip_ref[...], 0.0)
    gids = lax.broadcasted_iota(jnp.int32, (N, G), 1)
    m = (gids == seg_ref[...]).astype(jnp.float32)
    sums = lax.dot_general(m, h, (((0,), (0,)), ((), ())),
                           preferred_element_type=jnp.float32)
    counts = jnp.sum(m, axis=0)[:, None]
    pooled = sums / jnp.maximum(counts, 1.0)
    logits = (
        jnp.dot(pooled, wd_ref[...], preferred_element_type=jnp.float32)
        + bd_ref[...]
    )
    mx = jnp.max(logits, axis=1, keepdims=True)
    e = jnp.exp(logits - mx)
    out_ref[...] = e / jnp.sum(e, axis=1, keepdims=True)


_final_call = pl.pallas_call(
    _final_tc,
    out_shape=jax.ShapeDtypeStruct((G, L), jnp.float32),
)


def kernel(x, edge_index, i, W1_1, W2_1, b_1, W1_2, W2_2, b_2,
           W1_3, W2_3, b_3, Wd, bd):
    src = edge_index[0]
    dst = edge_index[1]
    pad = EPAD - E
    srcp = jnp.concatenate(
        [src, jnp.zeros((pad,), jnp.int32)]).reshape(NT, CPT, CH)
    dstp = jnp.concatenate(
        [dst, jnp.full((pad,), DUMMY, jnp.int32)]).reshape(NT, CPT, CH)
    ones8 = jnp.ones((CH, 8), jnp.float32)
    zeros8 = jnp.zeros((NP, 8), jnp.float32)
    zerosC = jnp.zeros((NP, C), jnp.float32)

    deg_t = _deg_call()(dstp, ones8, zeros8)
    hp, skip = _prep_call(deg_t, x, W1_1, W2_1, b_1.reshape(1, C))
    agg = _edge_call()(hp, srcp, dstp, zerosC)
    hp, skip = _mid_call(agg, deg_t, skip, W1_2, W2_2, b_2.reshape(1, C))
    agg = _edge_call()(hp, srcp, dstp, zerosC)
    hp, skip = _mid_call(agg, deg_t, skip, W1_3, W2_3, b_3.reshape(1, C))
    agg = _edge_call()(hp, srcp, dstp, zerosC)
    return _final_call(agg, deg_t, skip, i.reshape(N, 1), Wd,
                       bd.reshape(1, L))


# speedup vs baseline: 17.5929x; 1.1024x over previous
import functools

import jax
import jax.numpy as jnp
from jax import lax
from jax.experimental import pallas as pl
from jax.experimental.pallas import tpu as pltpu
from jax.experimental.pallas import tpu_sc as plsc

N = 10000
E = 320000
F = 128
C = 32
G = 16
L = 10

NC = 2
NS = 16
NT = NC * NS
CH = 128
CPT = 80
EPT = CH * CPT
EPAD = NT * EPT
NP = 10112
RPT = NP // NS
DUMMY = N


def _deg_body(dst_hbm, ones_hbm, zeros_hbm, out_hbm, dstv, onesv, deg_sh):
    c = lax.axis_index("c")
    s = lax.axis_index("s")
    wid = s * NC + c
    pltpu.sync_copy(dst_hbm.at[wid], dstv)
    pltpu.sync_copy(ones_hbm, onesv)
    pltpu.sync_copy(zeros_hbm.at[pl.ds(s * RPT, RPT)],
                    deg_sh.at[pl.ds(s * RPT, RPT)])
    plsc.subcore_barrier()

    def body(g, carry):
        pltpu.sync_copy(onesv, deg_sh.at[dstv.at[g]], add=True)
        return carry

    lax.fori_loop(0, CPT, body, 0)
    plsc.subcore_barrier()
    pltpu.sync_copy(deg_sh.at[pl.ds(s * RPT, RPT)],
                    out_hbm.at[c, pl.ds(s * RPT, RPT)])


@functools.cache
def _deg_call():
    return pl.kernel(
        _deg_body,
        out_type=jax.ShapeDtypeStruct((NC, NP, 8), jnp.float32),
        mesh=plsc.VectorSubcoreMesh(
            core_axis_name="c", subcore_axis_name="s",
            num_cores=NC, num_subcores=NS),
        scratch_types=[
            pltpu.VMEM((CPT, CH), jnp.int32),
            pltpu.VMEM((CH, 8), jnp.float32),
            pltpu.VMEM_SHARED((NP, 8), jnp.float32),
        ],
        compiler_params=pltpu.CompilerParams(use_tc_tiling_on_sc=False),
    )


NBUF = 4


def _edge_body(hp_hbm, src_hbm, dst_hbm, zeros_hbm, out_hbm,
               srcv, dstv, rows, gsems, ssems, agg_sh):
    c = lax.axis_index("c")
    s = lax.axis_index("s")
    wid = s * NC + c
    pltpu.sync_copy(src_hbm.at[wid], srcv)
    pltpu.sync_copy(dst_hbm.at[wid], dstv)
    pltpu.sync_copy(zeros_hbm.at[pl.ds(s * RPT, RPT)],
                    agg_sh.at[pl.ds(s * RPT, RPT)])
    plsc.subcore_barrier()

    def gather(g, j):
        pltpu.async_copy(hp_hbm.at[srcv.at[g]], rows.at[j], gsems.at[j])

    def scatter(g, j):
        pltpu.async_copy(rows.at[j], agg_sh.at[dstv.at[g]], ssems.at[j],
                         add=True)

    def wait_gather(g, j):
        pltpu.make_async_copy(hp_hbm.at[srcv.at[g]], rows.at[j],
                              gsems.at[j]).wait()

    def wait_scatter(g, j):
        pltpu.make_async_copy(rows.at[j], agg_sh.at[dstv.at[g]],
                              ssems.at[j]).wait()

    for j in range(NBUF):
        gather(j, j)

    def body(p, carry):
        for j in range(NBUF):
            g = NBUF * p + j
            wait_gather(g, j)
            scatter(g, j)
        for j in range(NBUF):
            g = NBUF * p + j
            wait_scatter(g, j)

            @pl.when(p + 1 < CPT // NBUF)
            def _():
                gather(g + NBUF, j)
        return carry

    lax.fori_loop(0, CPT // NBUF, body, 0)
    plsc.subcore_barrier()
    pltpu.sync_copy(agg_sh.at[pl.ds(s * RPT, RPT)],
                    out_hbm.at[c, pl.ds(s * RPT, RPT)])


@functools.cache
def _edge_call():
    return pl.kernel(
        _edge_body,
        out_type=jax.ShapeDtypeStruct((NC, NP, C), jnp.float32),
        mesh=plsc.VectorSubcoreMesh(
            core_axis_name="c", subcore_axis_name="s",
            num_cores=NC, num_subcores=NS),
        scratch_types=[
            pltpu.VMEM((CPT, CH), jnp.int32),
            pltpu.VMEM((CPT, CH), jnp.int32),
            pltpu.VMEM((NBUF, CH, C), jnp.float32),
            pltpu.SemaphoreType.DMA((NBUF,)),
            pltpu.SemaphoreType.DMA((NBUF,)),
            pltpu.VMEM_SHARED((NP, C), jnp.float32),
        ],
        compiler_params=pltpu.CompilerParams(use_tc_tiling_on_sc=False),
    )


def _dinv_col(deg_ref):
    deg = deg_ref[0] + deg_ref[1]
    return lax.rsqrt(jnp.maximum(deg, 1.0))[:N, 0:1]


def _prep_tc(deg_ref, x_ref, w1_ref, w2_ref, b_ref, hp_ref, skip_ref):
    dcol = _dinv_col(deg_ref)
    x = x_ref[...]
    h = jnp.dot(x, w1_ref[...], preferred_element_type=jnp.float32)
    hp_ref[...] = h * dcol
    skip_ref[...] = (
        jnp.dot(x, w2_ref[...], preferred_element_type=jnp.float32) + b_ref[...]
    )


_prep_call = pl.pallas_call(
    _prep_tc,
    out_shape=(
        jax.ShapeDtypeStruct((N, C), jnp.float32),
        jax.ShapeDtypeStruct((N, C), jnp.float32),
    ),
)


def _mid_tc(agg_ref, deg_ref, skip_ref, w1_ref, w2_ref, b_ref,
            hp_ref, skip2_ref):
    dcol = _dinv_col(deg_ref)
    agg = agg_ref[0, :N] + agg_ref[1, :N]
    h = jnp.maximum(agg * dcol + skip_ref[...], 0.0)
    hp_ref[...] = (
        jnp.dot(h, w1_ref[...], preferred_element_type=jnp.float32) * dcol
    )
    skip2_ref[...] = (
        jnp.dot(h, w2_ref[...], preferred_element_type=jnp.float32) + b_ref[...]
    )


_mid_call = pl.pallas_call(
    _mid_tc,
    out_shape=(
        jax.ShapeDtypeStruct((N, C), jnp.float32),
        jax.ShapeDtypeStruct((N, C), jnp.float32),
    ),
)


def _final_tc(agg_ref, deg_ref, skip_ref, seg_ref, wd_ref, bd_ref, out_ref):
    dcol = _dinv_col(deg_ref)
    agg = agg_ref[0, :N] + agg_ref[1, :N]
    h = jnp.maximum(agg * dcol + skip_ref[...], 0.0)
    gids = lax.broadcasted_iota(jnp.int32, (N, G), 1)
    m = (gids == seg_ref[...]).astype(jnp.float32)
    sums = lax.dot_general(m, h, (((0,), (0,)), ((), ())),
                           preferred_element_type=jnp.float32)
    counts = jnp.sum(m, axis=0)[:, None]
    pooled = sums / jnp.maximum(counts, 1.0)
    logits = (
        jnp.dot(pooled, wd_ref[...], preferred_element_type=jnp.float32)
        + bd_ref[...]
    )
    mx = jnp.max(logits, axis=1, keepdims=True)
    e = jnp.exp(logits - mx)
    out_ref[...] = e / jnp.sum(e, axis=1, keepdims=True)


_final_call = pl.pallas_call(
    _final_tc,
    out_shape=jax.ShapeDtypeStruct((G, L), jnp.float32),
)


def kernel(x, edge_index, i, W1_1, W2_1, b_1, W1_2, W2_2, b_2,
           W1_3, W2_3, b_3, Wd, bd):
    src = edge_index[0]
    dst = edge_index[1]
    pad = EPAD - E
    srcp = jnp.concatenate(
        [src, jnp.zeros((pad,), jnp.int32)]).reshape(NT, CPT, CH)
    dstp = jnp.concatenate(
        [dst, jnp.full((pad,), DUMMY, jnp.int32)]).reshape(NT, CPT, CH)
    ones8 = jnp.ones((CH, 8), jnp.float32)
    zeros8 = jnp.zeros((NP, 8), jnp.float32)
    zerosC = jnp.zeros((NP, C), jnp.float32)

    deg_t = _deg_call()(dstp, ones8, zeros8)
    hp, skip = _prep_call(deg_t, x, W1_1, W2_1, b_1.reshape(1, C))
    agg = _edge_call()(hp, srcp, dstp, zerosC)
    hp, skip = _mid_call(agg, deg_t, skip, W1_2, W2_2, b_2.reshape(1, C))
    agg = _edge_call()(hp, srcp, dstp, zerosC)
    hp, skip = _mid_call(agg, deg_t, skip, W1_3, W2_3, b_3.reshape(1, C))
    agg = _edge_call()(hp, srcp, dstp, zerosC)
    return _final_call(agg, deg_t, skip, i.reshape(N, 1), Wd,
                       bd.reshape(1, L))


# speedup vs baseline: 28.6989x; 1.6313x over previous
import functools

import jax
import jax.numpy as jnp
from jax import lax
from jax.experimental import pallas as pl
from jax.experimental.pallas import tpu as pltpu
from jax.experimental.pallas import tpu_sc as plsc

N = 10000
E = 320000
F = 128
C = 32
G = 16
L = 10

NC = 2
NS = 16
NT = NC * NS
CH = 128
CPT = 80
EPT = CH * CPT
EPAD = NT * EPT
NP = 10112
RPT = NP // NS
DUMMY = N


def _deg_body(dst_hbm, ones_hbm, zeros_hbm, out_hbm, dstv, onesv, deg_sh):
    c = lax.axis_index("c")
    s = lax.axis_index("s")
    wid = s * NC + c
    pltpu.sync_copy(dst_hbm.at[wid], dstv)
    pltpu.sync_copy(ones_hbm, onesv)
    pltpu.sync_copy(zeros_hbm.at[pl.ds(s * RPT, RPT)],
                    deg_sh.at[pl.ds(s * RPT, RPT)])
    plsc.subcore_barrier()

    def body(g, carry):
        pltpu.sync_copy(onesv, deg_sh.at[dstv.at[g]], add=True)
        return carry

    lax.fori_loop(0, CPT, body, 0)
    plsc.subcore_barrier()
    pltpu.sync_copy(deg_sh.at[pl.ds(s * RPT, RPT)],
                    out_hbm.at[c, pl.ds(s * RPT, RPT)])


@functools.cache
def _deg_call():
    return pl.kernel(
        _deg_body,
        out_type=jax.ShapeDtypeStruct((NC, NP, 8), jnp.float32),
        mesh=plsc.VectorSubcoreMesh(
            core_axis_name="c", subcore_axis_name="s",
            num_cores=NC, num_subcores=NS),
        scratch_types=[
            pltpu.VMEM((CPT, CH), jnp.int32),
            pltpu.VMEM((CH, 8), jnp.float32),
            pltpu.VMEM_SHARED((NP, 8), jnp.float32),
        ],
        compiler_params=pltpu.CompilerParams(use_tc_tiling_on_sc=False),
    )


NBUF = 4


def _edge_body(hp_hbm, src_hbm, dst_hbm, zeros_hbm, out_hbm,
               srcv, dstv, rows, gsems, ssems, agg_sh):
    c = lax.axis_index("c")
    s = lax.axis_index("s")
    wid = s * NC + c
    pltpu.sync_copy(src_hbm.at[wid], srcv)
    pltpu.sync_copy(dst_hbm.at[wid], dstv)
    pltpu.sync_copy(zeros_hbm.at[pl.ds(s * RPT, RPT)],
                    agg_sh.at[pl.ds(s * RPT, RPT)])
    plsc.subcore_barrier()

    def gather(g, j):
        pltpu.async_copy(hp_hbm.at[srcv.at[g]], rows.at[j], gsems.at[j])

    def scatter(g, j):
        pltpu.async_copy(rows.at[j], agg_sh.at[dstv.at[g]], ssems.at[j],
                         add=True)

    def wait_gather(g, j):
        pltpu.make_async_copy(hp_hbm.at[srcv.at[g]], rows.at[j],
                              gsems.at[j]).wait()

    def wait_scatter(g, j):
        pltpu.make_async_copy(rows.at[j], agg_sh.at[dstv.at[g]],
                              ssems.at[j]).wait()

    for j in range(NBUF):
        gather(j, j)

    def body(p, carry):
        for j in range(NBUF):
            g = NBUF * p + j
            wait_gather(g, j)
            scatter(g, j)
        for j in range(NBUF):
            g = NBUF * p + j
            wait_scatter(g, j)

            @pl.when(p + 1 < CPT // NBUF)
            def _():
                gather(g + NBUF, j)
        return carry

    lax.fori_loop(0, CPT // NBUF, body, 0)
    plsc.subcore_barrier()
    pltpu.sync_copy(agg_sh.at[pl.ds(s * RPT, RPT)],
                    out_hbm.at[c, pl.ds(s * RPT, RPT)])


@functools.cache
def _edge_call():
    return pl.kernel(
        _edge_body,
        out_type=jax.ShapeDtypeStruct((NC, NP, C), jnp.bfloat16),
        mesh=plsc.VectorSubcoreMesh(
            core_axis_name="c", subcore_axis_name="s",
            num_cores=NC, num_subcores=NS),
        scratch_types=[
            pltpu.VMEM((CPT, CH), jnp.int32),
            pltpu.VMEM((CPT, CH), jnp.int32),
            pltpu.VMEM((NBUF, CH, C), jnp.bfloat16),
            pltpu.SemaphoreType.DMA((NBUF,)),
            pltpu.SemaphoreType.DMA((NBUF,)),
            pltpu.VMEM_SHARED((NP, C), jnp.bfloat16),
        ],
        compiler_params=pltpu.CompilerParams(use_tc_tiling_on_sc=False),
    )


def _dinv_col(deg_ref):
    deg = deg_ref[0] + deg_ref[1]
    return lax.rsqrt(jnp.maximum(deg, 1.0))[:N, 0:1]


def _prep_tc(deg_ref, x_ref, w1_ref, w2_ref, b_ref, hp_ref, skip_ref):
    dcol = _dinv_col(deg_ref)
    x = x_ref[...]
    h = jnp.dot(x, w1_ref[...], preferred_element_type=jnp.float32)
    hp_ref[...] = (h * dcol).astype(jnp.bfloat16)
    skip_ref[...] = (
        jnp.dot(x, w2_ref[...], preferred_element_type=jnp.float32) + b_ref[...]
    )


_prep_call = pl.pallas_call(
    _prep_tc,
    out_shape=(
        jax.ShapeDtypeStruct((N, C), jnp.bfloat16),
        jax.ShapeDtypeStruct((N, C), jnp.float32),
    ),
)


def _mid_tc(agg_ref, deg_ref, skip_ref, w1_ref, w2_ref, b_ref,
            hp_ref, skip2_ref):
    dcol = _dinv_col(deg_ref)
    agg = (agg_ref[0, :N].astype(jnp.float32)
           + agg_ref[1, :N].astype(jnp.float32))
    h = jnp.maximum(agg * dcol + skip_ref[...], 0.0)
    hp_ref[...] = (
        jnp.dot(h, w1_ref[...], preferred_element_type=jnp.float32) * dcol
    ).astype(jnp.bfloat16)
    skip2_ref[...] = (
        jnp.dot(h, w2_ref[...], preferred_element_type=jnp.float32) + b_ref[...]
    )


_mid_call = pl.pallas_call(
    _mid_tc,
    out_shape=(
        jax.ShapeDtypeStruct((N, C), jnp.bfloat16),
        jax.ShapeDtypeStruct((N, C), jnp.float32),
    ),
)


def _final_tc(agg_ref, deg_ref, skip_ref, seg_ref, wd_ref, bd_ref, out_ref):
    dcol = _dinv_col(deg_ref)
    agg = (agg_ref[0, :N].astype(jnp.float32)
           + agg_ref[1, :N].astype(jnp.float32))
    h = jnp.maximum(agg * dcol + skip_ref[...], 0.0)
    gids = lax.broadcasted_iota(jnp.int32, (N, G), 1)
    m = (gids == seg_ref[...]).astype(jnp.float32)
    sums = lax.dot_general(m, h, (((0,), (0,)), ((), ())),
                           preferred_element_type=jnp.float32)
    counts = jnp.sum(m, axis=0)[:, None]
    pooled = sums / jnp.maximum(counts, 1.0)
    logits = (
        jnp.dot(pooled, wd_ref[...], preferred_element_type=jnp.float32)
        + bd_ref[...]
    )
    mx = jnp.max(logits, axis=1, keepdims=True)
    e = jnp.exp(logits - mx)
    out_ref[...] = e / jnp.sum(e, axis=1, keepdims=True)


_final_call = pl.pallas_call(
    _final_tc,
    out_shape=jax.ShapeDtypeStruct((G, L), jnp.float32),
)


def kernel(x, edge_index, i, W1_1, W2_1, b_1, W1_2, W2_2, b_2,
           W1_3, W2_3, b_3, Wd, bd):
    src = edge_index[0]
    dst = edge_index[1]
    pad = EPAD - E
    srcp = jnp.concatenate(
        [src, jnp.zeros((pad,), jnp.int32)]).reshape(NT, CPT, CH)
    dstp = jnp.concatenate(
        [dst, jnp.full((pad,), DUMMY, jnp.int32)]).reshape(NT, CPT, CH)
    ones8 = jnp.ones((CH, 8), jnp.float32)
    zeros8 = jnp.zeros((NP, 8), jnp.float32)
    zerosC = jnp.zeros((NP, C), jnp.bfloat16)

    deg_t = _deg_call()(dstp, ones8, zeros8)
    hp, skip = _prep_call(deg_t, x, W1_1, W2_1, b_1.reshape(1, C))
    agg = _edge_call()(hp, srcp, dstp, zerosC)
    hp, skip = _mid_call(agg, deg_t, skip, W1_2, W2_2, b_2.reshape(1, C))
    agg = _edge_call()(hp, srcp, dstp, zerosC)
    hp, skip = _mid_call(agg, deg_t, skip, W1_3, W2_3, b_3.reshape(1, C))
    agg = _edge_call()(hp, srcp, dstp, zerosC)
    return _final_call(agg, deg_t, skip, i.reshape(N, 1), Wd,
                       bd.reshape(1, L))


# speedup vs baseline: 43.7546x; 1.5246x over previous
import functools

import jax
import jax.numpy as jnp
from jax import lax
from jax.experimental import pallas as pl
from jax.experimental.pallas import tpu as pltpu
from jax.experimental.pallas import tpu_sc as plsc

N = 10000
E = 320000
F = 128
C = 32
G = 16
L = 10

NC = 2
NS = 16
NT = NC * NS
CH = 128
NCH = E // CH
CPT = NCH // NT + 1
NP = 10112
RPT = NP // NS


def _tile_span(wid):
    start = (NCH * wid) // NT
    cnt = (NCH * (wid + 1)) // NT - start
    return start, cnt


def _deg_body(ei_hbm, ones_hbm, zeros_hbm, out_hbm, dstv, onesv, deg_sh):
    c = lax.axis_index("c")
    s = lax.axis_index("s")
    wid = s * NC + c
    start, cnt = _tile_span(wid)
    pltpu.sync_copy(ei_hbm.at[1, pl.ds(start, CPT)], dstv)
    pltpu.sync_copy(ones_hbm, onesv)
    pltpu.sync_copy(zeros_hbm.at[pl.ds(s * RPT, RPT)],
                    deg_sh.at[pl.ds(s * RPT, RPT)])
    plsc.subcore_barrier()

    def body(g, carry):
        pltpu.sync_copy(onesv, deg_sh.at[dstv.at[g]], add=True)
        return carry

    lax.fori_loop(0, cnt, body, 0)
    plsc.subcore_barrier()
    pltpu.sync_copy(deg_sh.at[pl.ds(s * RPT, RPT)],
                    out_hbm.at[c, pl.ds(s * RPT, RPT)])


@functools.cache
def _deg_call():
    return pl.kernel(
        _deg_body,
        out_type=jax.ShapeDtypeStruct((NC, NP, 8), jnp.float32),
        mesh=plsc.VectorSubcoreMesh(
            core_axis_name="c", subcore_axis_name="s",
            num_cores=NC, num_subcores=NS),
        scratch_types=[
            pltpu.VMEM((CPT, CH), jnp.int32),
            pltpu.VMEM((CH, 8), jnp.float32),
            pltpu.VMEM_SHARED((NP, 8), jnp.float32),
        ],
        compiler_params=pltpu.CompilerParams(use_tc_tiling_on_sc=False),
    )


NBUF = 4


FULL = 76


def _edge_body(hp_hbm, ei_hbm, zeros_hbm, out_hbm,
               srcv, dstv, rows, gsems, ssems, agg_sh):
    c = lax.axis_index("c")
    s = lax.axis_index("s")
    wid = s * NC + c
    start, cnt = _tile_span(wid)
    pltpu.sync_copy(ei_hbm.at[0, pl.ds(start, CPT)], srcv)
    pltpu.sync_copy(ei_hbm.at[1, pl.ds(start, CPT)], dstv)
    pltpu.sync_copy(zeros_hbm.at[pl.ds(s * RPT, RPT)],
                    agg_sh.at[pl.ds(s * RPT, RPT)])
    plsc.subcore_barrier()

    def gather(g, j):
        pltpu.async_copy(hp_hbm.at[srcv.at[g]], rows.at[j], gsems.at[j])

    def scatter(g, j):
        pltpu.async_copy(rows.at[j], agg_sh.at[dstv.at[g]], ssems.at[j],
                         add=True)

    def wait_gather(g, j):
        pltpu.make_async_copy(hp_hbm.at[srcv.at[g]], rows.at[j],
                              gsems.at[j]).wait()

    def wait_scatter(g, j):
        pltpu.make_async_copy(rows.at[j], agg_sh.at[dstv.at[g]],
                              ssems.at[j]).wait()

    for j in range(NBUF):
        gather(j, j)

    def body(p, carry):
        for j in range(NBUF):
            g = NBUF * p + j
            wait_gather(g, j)
            scatter(g, j)
        for j in range(NBUF):
            g = NBUF * p + j
            wait_scatter(g, j)
            nxt = g + NBUF

            @pl.when(nxt < cnt)
            def _():
                gather(nxt, j)
        return carry

    lax.fori_loop(0, FULL // NBUF, body, 0)
    for j in range(NBUF):
        g = FULL + j

        @pl.when(g < cnt)
        def _():
            wait_gather(g, j)
            scatter(g, j)
            wait_scatter(g, j)

    plsc.subcore_barrier()
    pltpu.sync_copy(agg_sh.at[pl.ds(s * RPT, RPT)],
                    out_hbm.at[c, pl.ds(s * RPT, RPT)])


@functools.cache
def _edge_call():
    return pl.kernel(
        _edge_body,
        out_type=jax.ShapeDtypeStruct((NC, NP, C), jnp.bfloat16),
        mesh=plsc.VectorSubcoreMesh(
            core_axis_name="c", subcore_axis_name="s",
            num_cores=NC, num_subcores=NS),
        scratch_types=[
            pltpu.VMEM((CPT, CH), jnp.int32),
            pltpu.VMEM((CPT, CH), jnp.int32),
            pltpu.VMEM((NBUF, CH, C), jnp.bfloat16),
            pltpu.SemaphoreType.DMA((NBUF,)),
            pltpu.SemaphoreType.DMA((NBUF,)),
            pltpu.VMEM_SHARED((NP, C), jnp.bfloat16),
        ],
        compiler_params=pltpu.CompilerParams(use_tc_tiling_on_sc=False),
    )


def _dinv_col(deg_ref):
    deg = deg_ref[0] + deg_ref[1]
    return lax.rsqrt(jnp.maximum(deg, 1.0))[:N, 0:1]


def _prep_tc(deg_ref, x_ref, w1_ref, w2_ref, b_ref, hp_ref, skip_ref):
    dcol = _dinv_col(deg_ref)
    x = x_ref[...]
    h = jnp.dot(x, w1_ref[...], preferred_element_type=jnp.float32)
    hp_ref[...] = (h * dcol).astype(jnp.bfloat16)
    skip_ref[...] = (
        jnp.dot(x, w2_ref[...], preferred_element_type=jnp.float32) + b_ref[...]
    )


_prep_call = pl.pallas_call(
    _prep_tc,
    out_shape=(
        jax.ShapeDtypeStruct((N, C), jnp.bfloat16),
        jax.ShapeDtypeStruct((N, C), jnp.float32),
    ),
)


def _mid_tc(agg_ref, deg_ref, skip_ref, w1_ref, w2_ref, b_ref,
            hp_ref, skip2_ref):
    dcol = _dinv_col(deg_ref)
    agg = (agg_ref[0, :N].astype(jnp.float32)
           + agg_ref[1, :N].astype(jnp.float32))
    h = jnp.maximum(agg * dcol + skip_ref[...], 0.0)
    hp_ref[...] = (
        jnp.dot(h, w1_ref[...], preferred_element_type=jnp.float32) * dcol
    ).astype(jnp.bfloat16)
    skip2_ref[...] = (
        jnp.dot(h, w2_ref[...], preferred_element_type=jnp.float32) + b_ref[...]
    )


_mid_call = pl.pallas_call(
    _mid_tc,
    out_shape=(
        jax.ShapeDtypeStruct((N, C), jnp.bfloat16),
        jax.ShapeDtypeStruct((N, C), jnp.float32),
    ),
)


def _final_tc(agg_ref, deg_ref, skip_ref, seg_ref, wd_ref, bd_ref, out_ref):
    dcol = _dinv_col(deg_ref)
    agg = (agg_ref[0, :N].astype(jnp.float32)
           + agg_ref[1, :N].astype(jnp.float32))
    h = jnp.maximum(agg * dcol + skip_ref[...], 0.0)
    gids = lax.broadcasted_iota(jnp.int32, (N, G), 1)
    m = (gids == seg_ref[...]).astype(jnp.float32)
    sums = lax.dot_general(m, h, (((0,), (0,)), ((), ())),
                           preferred_element_type=jnp.float32)
    counts = jnp.sum(m, axis=0)[:, None]
    pooled = sums / jnp.maximum(counts, 1.0)
    logits = (
        jnp.dot(pooled, wd_ref[...], preferred_element_type=jnp.float32)
        + bd_ref[...]
    )
    mx = jnp.max(logits, axis=1, keepdims=True)
    e = jnp.exp(logits - mx)
    out_ref[...] = e / jnp.sum(e, axis=1, keepdims=True)


_final_call = pl.pallas_call(
    _final_tc,
    out_shape=jax.ShapeDtypeStruct((G, L), jnp.float32),
)


def kernel(x, edge_index, i, W1_1, W2_1, b_1, W1_2, W2_2, b_2,
           W1_3, W2_3, b_3, Wd, bd):
    ei3 = edge_index.reshape(2, NCH, CH)
    ones8 = jnp.ones((CH, 8), jnp.float32)
    zeros8 = jnp.zeros((NP, 8), jnp.float32)
    zerosC = jnp.zeros((NP, C), jnp.bfloat16)

    deg_t = _deg_call()(ei3, ones8, zeros8)
    hp, skip = _prep_call(deg_t, x, W1_1, W2_1, b_1.reshape(1, C))
    agg = _edge_call()(hp, ei3, zerosC)
    hp, skip = _mid_call(agg, deg_t, skip, W1_2, W2_2, b_2.reshape(1, C))
    agg = _edge_call()(hp, ei3, zerosC)
    hp, skip = _mid_call(agg, deg_t, skip, W1_3, W2_3, b_3.reshape(1, C))
    agg = _edge_call()(hp, ei3, zerosC)
    return _final_call(agg, deg_t, skip, i.reshape(N, 1), Wd,
                       bd.reshape(1, L))


# speedup vs baseline: 46.0077x; 1.0515x over previous
import functools

import jax
import jax.numpy as jnp
from jax import lax
from jax.experimental import pallas as pl
from jax.experimental.pallas import tpu as pltpu
from jax.experimental.pallas import tpu_sc as plsc

N = 10000
E = 320000
F = 128
C = 32
G = 16
L = 10

NC = 2
NS = 16
NT = NC * NS
CH = 128
NCH = E // CH
CPT = NCH // NT + 1
NP = 10112
RPT = NP // NS


def _tile_span(wid):
    start = (NCH * wid) // NT
    cnt = (NCH * (wid + 1)) // NT - start
    return start, cnt


def _deg_body(ei_hbm, ones_hbm, zeros_hbm, out_hbm, dstv, onesv, deg_sh):
    c = lax.axis_index("c")
    s = lax.axis_index("s")
    wid = s * NC + c
    start, cnt = _tile_span(wid)
    pltpu.sync_copy(ei_hbm.at[1, pl.ds(start, CPT)], dstv)
    pltpu.sync_copy(ones_hbm, onesv)
    pltpu.sync_copy(zeros_hbm.at[pl.ds(s * RPT, RPT)],
                    deg_sh.at[pl.ds(s * RPT, RPT)])
    plsc.subcore_barrier()

    def body(g, carry):
        pltpu.sync_copy(onesv, deg_sh.at[dstv.at[g]], add=True)
        return carry

    lax.fori_loop(0, cnt, body, 0)
    plsc.subcore_barrier()
    pltpu.sync_copy(deg_sh.at[pl.ds(s * RPT, RPT)],
                    out_hbm.at[c, pl.ds(s * RPT, RPT)])


@functools.cache
def _deg_call():
    return pl.kernel(
        _deg_body,
        out_type=jax.ShapeDtypeStruct((NC, NP, 8), jnp.float32),
        mesh=plsc.VectorSubcoreMesh(
            core_axis_name="c", subcore_axis_name="s",
            num_cores=NC, num_subcores=NS),
        scratch_types=[
            pltpu.VMEM((CPT, CH), jnp.int32),
            pltpu.VMEM((CH, 8), jnp.float32),
            pltpu.VMEM_SHARED((NP, 8), jnp.float32),
        ],
        compiler_params=pltpu.CompilerParams(use_tc_tiling_on_sc=False),
    )


NBUF = 8


FULL = 72


def _edge_body(hp_hbm, ei_hbm, zeros_hbm, out_hbm,
               srcv, dstv, rows, gsems, ssems, agg_sh):
    c = lax.axis_index("c")
    s = lax.axis_index("s")
    wid = s * NC + c
    start, cnt = _tile_span(wid)
    pltpu.sync_copy(ei_hbm.at[0, pl.ds(start, CPT)], srcv)
    pltpu.sync_copy(ei_hbm.at[1, pl.ds(start, CPT)], dstv)
    pltpu.sync_copy(zeros_hbm.at[pl.ds(s * RPT, RPT)],
                    agg_sh.at[pl.ds(s * RPT, RPT)])
    plsc.subcore_barrier()

    def gather(g, j):
        pltpu.async_copy(hp_hbm.at[srcv.at[g]], rows.at[j], gsems.at[j])

    def scatter(g, j):
        pltpu.async_copy(rows.at[j], agg_sh.at[dstv.at[g]], ssems.at[j],
                         add=True)

    def wait_gather(g, j):
        pltpu.make_async_copy(hp_hbm.at[srcv.at[g]], rows.at[j],
                              gsems.at[j]).wait()

    def wait_scatter(g, j):
        pltpu.make_async_copy(rows.at[j], agg_sh.at[dstv.at[g]],
                              ssems.at[j]).wait()

    for j in range(NBUF):
        gather(j, j)

    def body(p, carry):
        for j in range(NBUF):
            g = NBUF * p + j
            wait_gather(g, j)
            scatter(g, j)
        for j in range(NBUF):
            g = NBUF * p + j
            wait_scatter(g, j)
            nxt = g + NBUF

            @pl.when(nxt < cnt)
            def _():
                gather(nxt, j)
        return carry

    lax.fori_loop(0, FULL // NBUF, body, 0)
    for j in range(NBUF):
        g = FULL + j

        @pl.when(g < cnt)
        def _():
            wait_gather(g, j)
            scatter(g, j)
            wait_scatter(g, j)

    plsc.subcore_barrier()
    pltpu.sync_copy(agg_sh.at[pl.ds(s * RPT, RPT)],
                    out_hbm.at[c, pl.ds(s * RPT, RPT)])


@functools.cache
def _edge_call():
    return pl.kernel(
        _edge_body,
        out_type=jax.ShapeDtypeStruct((NC, NP, C), jnp.bfloat16),
        mesh=plsc.VectorSubcoreMesh(
            core_axis_name="c", subcore_axis_name="s",
            num_cores=NC, num_subcores=NS),
        scratch_types=[
            pltpu.VMEM((CPT, CH), jnp.int32),
            pltpu.VMEM((CPT, CH), jnp.int32),
            pltpu.VMEM((NBUF, CH, C), jnp.bfloat16),
            pltpu.SemaphoreType.DMA((NBUF,)),
            pltpu.SemaphoreType.DMA((NBUF,)),
            pltpu.VMEM_SHARED((NP, C), jnp.bfloat16),
        ],
        compiler_params=pltpu.CompilerParams(use_tc_tiling_on_sc=False),
    )


NB = 10
BR = N // NB

_deg_spec = pl.BlockSpec((NC, BR, 8), lambda i: (0, i, 0))
_row_spec = pl.BlockSpec((BR, C), lambda i: (i, 0))
_agg_spec = pl.BlockSpec((NC, BR, C), lambda i: (0, i, 0))


def _full(shape):
    return pl.BlockSpec(shape, lambda i: tuple(0 for _ in shape))


def _dinv_col(deg_ref):
    deg = deg_ref[0] + deg_ref[1]
    return lax.rsqrt(jnp.maximum(deg, 1.0))[:, 0:1]


def _prep_tc(deg_ref, x_ref, w1_ref, w2_ref, b_ref, hp_ref, skip_ref):
    dcol = _dinv_col(deg_ref)
    x = x_ref[...]
    h = jnp.dot(x, w1_ref[...], preferred_element_type=jnp.float32)
    hp_ref[...] = (h * dcol).astype(jnp.bfloat16)
    skip_ref[...] = (
        jnp.dot(x, w2_ref[...], preferred_element_type=jnp.float32) + b_ref[...]
    )


_prep_call = pl.pallas_call(
    _prep_tc,
    grid=(NB,),
    in_specs=[_deg_spec, pl.BlockSpec((BR, F), lambda i: (i, 0)),
              _full((F, C)), _full((F, C)), _full((1, C))],
    out_specs=(_row_spec, _row_spec),
    out_shape=(
        jax.ShapeDtypeStruct((N, C), jnp.bfloat16),
        jax.ShapeDtypeStruct((N, C), jnp.float32),
    ),
)


def _mid_tc(agg_ref, deg_ref, skip_ref, w1_ref, w2_ref, b_ref,
            hp_ref, skip2_ref):
    dcol = _dinv_col(deg_ref)
    agg = (agg_ref[0].astype(jnp.float32)
           + agg_ref[1].astype(jnp.float32))
    h = jnp.maximum(agg * dcol + skip_ref[...], 0.0)
    hp_ref[...] = (
        jnp.dot(h, w1_ref[...], preferred_element_type=jnp.float32) * dcol
    ).astype(jnp.bfloat16)
    skip2_ref[...] = (
        jnp.dot(h, w2_ref[...], preferred_element_type=jnp.float32) + b_ref[...]
    )


_mid_call = pl.pallas_call(
    _mid_tc,
    grid=(NB,),
    in_specs=[_agg_spec, _deg_spec, _row_spec,
              _full((C, C)), _full((C, C)), _full((1, C))],
    out_specs=(_row_spec, _row_spec),
    out_shape=(
        jax.ShapeDtypeStruct((N, C), jnp.bfloat16),
        jax.ShapeDtypeStruct((N, C), jnp.float32),
    ),
)


def _final_tc(agg_ref, deg_ref, skip_ref, seg_ref, wd_ref, bd_ref, out_ref,
              sums_ref, counts_ref):
    i = pl.program_id(0)
    dcol = _dinv_col(deg_ref)
    agg = (agg_ref[0].astype(jnp.float32)
           + agg_ref[1].astype(jnp.float32))
    h = jnp.maximum(agg * dcol + skip_ref[...], 0.0)
    gids = lax.broadcasted_iota(jnp.int32, (BR, G), 1)
    m = (gids == seg_ref[...]).astype(jnp.float32)
    sums = lax.dot_general(m, h, (((0,), (0,)), ((), ())),
                           preferred_element_type=jnp.float32)
    counts = jnp.sum(m, axis=0, keepdims=True)

    @pl.when(i == 0)
    def _():
        sums_ref[...] = jnp.zeros_like(sums_ref)
        counts_ref[...] = jnp.zeros_like(counts_ref)

    sums_ref[...] += sums
    counts_ref[...] += counts

    @pl.when(i == NB - 1)
    def _():
        pooled = sums_ref[...] / jnp.maximum(counts_ref[...], 1.0).T
        logits = (
            jnp.dot(pooled, wd_ref[...], preferred_element_type=jnp.float32)
            + bd_ref[...]
        )
        mx = jnp.max(logits, axis=1, keepdims=True)
        e = jnp.exp(logits - mx)
        out_ref[...] = e / jnp.sum(e, axis=1, keepdims=True)


_final_call = pl.pallas_call(
    _final_tc,
    grid=(NB,),
    in_specs=[_agg_spec, _deg_spec, _row_spec,
              pl.BlockSpec((BR, 1), lambda i: (i, 0)),
              _full((C, L)), _full((1, L))],
    out_specs=pl.BlockSpec((G, L), lambda i: (0, 0)),
    out_shape=jax.ShapeDtypeStruct((G, L), jnp.float32),
    scratch_shapes=[pltpu.VMEM((G, C), jnp.float32),
                    pltpu.VMEM((1, G), jnp.float32)],
)


def kernel(x, edge_index, i, W1_1, W2_1, b_1, W1_2, W2_2, b_2,
           W1_3, W2_3, b_3, Wd, bd):
    ei3 = edge_index.reshape(2, NCH, CH)
    ones8 = jnp.ones((CH, 8), jnp.float32)
    zeros8 = jnp.zeros((NP, 8), jnp.float32)
    zerosC = jnp.zeros((NP, C), jnp.bfloat16)

    deg_t = _deg_call()(ei3, ones8, zeros8)
    hp, skip = _prep_call(deg_t, x, W1_1, W2_1, b_1.reshape(1, C))
    agg = _edge_call()(hp, ei3, zerosC)
    hp, skip = _mid_call(agg, deg_t, skip, W1_2, W2_2, b_2.reshape(1, C))
    agg = _edge_call()(hp, ei3, zerosC)
    hp, skip = _mid_call(agg, deg_t, skip, W1_3, W2_3, b_3.reshape(1, C))
    agg = _edge_call()(hp, ei3, zerosC)
    return _final_call(agg, deg_t, skip, i.reshape(N, 1), Wd,
                       bd.reshape(1, L))


# speedup vs baseline: 48.7569x; 1.0598x over previous
import functools

import jax
import jax.numpy as jnp
from jax import lax
from jax.experimental import pallas as pl
from jax.experimental.pallas import tpu as pltpu
from jax.experimental.pallas import tpu_sc as plsc

N = 10000
E = 320000
F = 128
C = 32
G = 16
L = 10

NC = 2
NS = 16
NT = NC * NS
CH = 128
NCH = E // CH
CPT = NCH // NT + 1
NP = 10112
RPT = NP // NS


def _tile_span(wid):
    start = (NCH * wid) // NT
    cnt = (NCH * (wid + 1)) // NT - start
    return start, cnt


def _deg_body(ei_hbm, ones_hbm, zeros_hbm, out_hbm, dstv, onesv, deg_sh):
    c = lax.axis_index("c")
    s = lax.axis_index("s")
    wid = s * NC + c
    start, cnt = _tile_span(wid)
    pltpu.sync_copy(ei_hbm.at[1, pl.ds(start, CPT)], dstv)
    pltpu.sync_copy(ones_hbm, onesv)
    pltpu.sync_copy(zeros_hbm.at[pl.ds(s * RPT, RPT)],
                    deg_sh.at[pl.ds(s * RPT, RPT)])
    plsc.subcore_barrier()

    def body(g, carry):
        pltpu.sync_copy(onesv, deg_sh.at[dstv.at[g]], add=True)
        return carry

    lax.fori_loop(0, cnt, body, 0)
    plsc.subcore_barrier()
    pltpu.sync_copy(deg_sh.at[pl.ds(s * RPT, RPT)],
                    out_hbm.at[c, pl.ds(s * RPT, RPT)])


@functools.cache
def _deg_call():
    return pl.kernel(
        _deg_body,
        out_type=jax.ShapeDtypeStruct((NC, NP, 8), jnp.float32),
        mesh=plsc.VectorSubcoreMesh(
            core_axis_name="c", subcore_axis_name="s",
            num_cores=NC, num_subcores=NS),
        scratch_types=[
            pltpu.VMEM((CPT, CH), jnp.int32),
            pltpu.VMEM((CH, 8), jnp.float32),
            pltpu.VMEM_SHARED((NP, 8), jnp.float32),
        ],
        compiler_params=pltpu.CompilerParams(use_tc_tiling_on_sc=False),
    )


NBUF = 8


FULL = 72


def _edge_body(hp_hbm, ei_hbm, zeros_hbm, out_hbm,
               srcv, dstv, rows, gsems, ssems, agg_sh):
    c = lax.axis_index("c")
    s = lax.axis_index("s")
    wid = s * NC + c
    start, cnt = _tile_span(wid)
    pltpu.sync_copy(ei_hbm.at[0, pl.ds(start, CPT)], srcv)
    pltpu.sync_copy(ei_hbm.at[1, pl.ds(start, CPT)], dstv)
    pltpu.sync_copy(zeros_hbm.at[pl.ds(s * RPT, RPT)],
                    agg_sh.at[pl.ds(s * RPT, RPT)])
    plsc.subcore_barrier()

    def gather(g, j):
        pltpu.async_copy(hp_hbm.at[srcv.at[g]], rows.at[j], gsems.at[j])

    def scatter(g, j):
        pltpu.async_copy(rows.at[j], agg_sh.at[dstv.at[g]], ssems.at[j],
                         add=True)

    def wait_gather(g, j):
        pltpu.make_async_copy(hp_hbm.at[srcv.at[g]], rows.at[j],
                              gsems.at[j]).wait()

    def wait_scatter(g, j):
        pltpu.make_async_copy(rows.at[j], agg_sh.at[dstv.at[g]],
                              ssems.at[j]).wait()

    for j in range(NBUF):
        gather(j, j)

    def body(p, carry):
        for j in range(NBUF):
            g = NBUF * p + j
            wait_gather(g, j)
            scatter(g, j)
        for j in range(NBUF):
            g = NBUF * p + j
            wait_scatter(g, j)
            nxt = g + NBUF

            @pl.when(nxt < cnt)
            def _():
                gather(nxt, j)
        return carry

    lax.fori_loop(0, FULL // NBUF, body, 0)
    for j in range(NBUF):
        g = FULL + j

        @pl.when(g < cnt)
        def _():
            wait_gather(g, j)
            scatter(g, j)
            wait_scatter(g, j)

    plsc.subcore_barrier()
    pltpu.sync_copy(agg_sh.at[pl.ds(s * RPT, RPT)],
                    out_hbm.at[c, pl.ds(s * RPT, RPT)])


@functools.cache
def _edge_call():
    return pl.kernel(
        _edge_body,
        out_type=jax.ShapeDtypeStruct((NC, NP, C), jnp.bfloat16),
        mesh=plsc.VectorSubcoreMesh(
            core_axis_name="c", subcore_axis_name="s",
            num_cores=NC, num_subcores=NS),
        scratch_types=[
            pltpu.VMEM((CPT, CH), jnp.int32),
            pltpu.VMEM((CPT, CH), jnp.int32),
            pltpu.VMEM((NBUF, CH, C), jnp.bfloat16),
            pltpu.SemaphoreType.DMA((NBUF,)),
            pltpu.SemaphoreType.DMA((NBUF,)),
            pltpu.VMEM_SHARED((NP, C), jnp.bfloat16),
        ],
        compiler_params=pltpu.CompilerParams(use_tc_tiling_on_sc=False),
    )


def _dinv_col(deg_ref):
    deg = deg_ref[0] + deg_ref[1]
    return lax.rsqrt(jnp.maximum(deg, 1.0))[:N, 0:1]


def _prep_tc(deg_ref, x_ref, w1_ref, w2_ref, b_ref, hp_ref, skip_ref):
    dcol = _dinv_col(deg_ref)
    x = x_ref[...]
    h = jnp.dot(x, w1_ref[...], preferred_element_type=jnp.float32)
    hp_ref[...] = (h * dcol).astype(jnp.bfloat16)
    skip_ref[...] = (
        jnp.dot(x, w2_ref[...], preferred_element_type=jnp.float32) + b_ref[...]
    ).astype(jnp.bfloat16)


_prep_call = pl.pallas_call(
    _prep_tc,
    out_shape=(
        jax.ShapeDtypeStruct((N, C), jnp.bfloat16),
        jax.ShapeDtypeStruct((N, C), jnp.bfloat16),
    ),
)


def _mid_tc(agg_ref, deg_ref, skip_ref, w1_ref, w2_ref, b_ref,
            hp_ref, skip2_ref):
    dcol = _dinv_col(deg_ref)
    agg = (agg_ref[0, :N].astype(jnp.float32)
           + agg_ref[1, :N].astype(jnp.float32))
    h = jnp.maximum(agg * dcol + skip_ref[...].astype(jnp.float32), 0.0)
    hp_ref[...] = (
        jnp.dot(h, w1_ref[...], preferred_element_type=jnp.float32) * dcol
    ).astype(jnp.bfloat16)
    skip2_ref[...] = (
        jnp.dot(h, w2_ref[...], preferred_element_type=jnp.float32) + b_ref[...]
    ).astype(jnp.bfloat16)


_mid_call = pl.pallas_call(
    _mid_tc,
    out_shape=(
        jax.ShapeDtypeStruct((N, C), jnp.bfloat16),
        jax.ShapeDtypeStruct((N, C), jnp.bfloat16),
    ),
)


def _final_tc(agg_ref, deg_ref, skip_ref, seg_ref, wd_ref, bd_ref, out_ref):
    dcol = _dinv_col(deg_ref)
    agg = (agg_ref[0, :N].astype(jnp.float32)
           + agg_ref[1, :N].astype(jnp.float32))
    h = jnp.maximum(agg * dcol + skip_ref[...].astype(jnp.float32), 0.0)
    gids = lax.broadcasted_iota(jnp.int32, (N, G), 1)
    m = (gids == seg_ref[...]).astype(jnp.float32)
    sums = lax.dot_general(m, h, (((0,), (0,)), ((), ())),
                           preferred_element_type=jnp.float32)
    counts = jnp.sum(m, axis=0)[:, None]
    pooled = sums / jnp.maximum(counts, 1.0)
    logits = (
        jnp.dot(pooled, wd_ref[...], preferred_element_type=jnp.float32)
        + bd_ref[...]
    )
    mx = jnp.max(logits, axis=1, keepdims=True)
    e = jnp.exp(logits - mx)
    out_ref[...] = e / jnp.sum(e, axis=1, keepdims=True)


_final_call = pl.pallas_call(
    _final_tc,
    out_shape=jax.ShapeDtypeStruct((G, L), jnp.float32),
)


def kernel(x, edge_index, i, W1_1, W2_1, b_1, W1_2, W2_2, b_2,
           W1_3, W2_3, b_3, Wd, bd):
    ei3 = edge_index.reshape(2, NCH, CH)
    ones8 = jnp.ones((CH, 8), jnp.float32)
    zeros8 = jnp.zeros((NP, 8), jnp.float32)
    zerosC = jnp.zeros((NP, C), jnp.bfloat16)

    deg_t = _deg_call()(ei3, ones8, zeros8)
    hp, skip = _prep_call(deg_t, x, W1_1, W2_1, b_1.reshape(1, C))
    agg = _edge_call()(hp, ei3, zerosC)
    hp, skip = _mid_call(agg, deg_t, skip, W1_2, W2_2, b_2.reshape(1, C))
    agg = _edge_call()(hp, ei3, zerosC)
    hp, skip = _mid_call(agg, deg_t, skip, W1_3, W2_3, b_3.reshape(1, C))
    agg = _edge_call()(hp, ei3, zerosC)
    return _final_call(agg, deg_t, skip, i.reshape(N, 1), Wd,
                       bd.reshape(1, L))


# speedup vs baseline: 50.1160x; 1.0279x over previous
import functools

import jax
import jax.numpy as jnp
from jax import lax
from jax.experimental import pallas as pl
from jax.experimental.pallas import tpu as pltpu
from jax.experimental.pallas import tpu_sc as plsc

N = 10000
E = 320000
F = 128
C = 32
G = 16
L = 10

NC = 2
NS = 16
NT = NC * NS
CH = 128
NCH = E // CH
CPT = NCH // NT + 1
NP = 10112
RPT = NP // NS


def _tile_span(wid):
    start = (NCH * wid) // NT
    cnt = (NCH * (wid + 1)) // NT - start
    return start, cnt


def _deg_body(ei_hbm, ones_hbm, zeros_hbm, out_hbm, dstv, onesv, deg_sh):
    c = lax.axis_index("c")
    s = lax.axis_index("s")
    wid = s * NC + c
    start, cnt = _tile_span(wid)
    pltpu.sync_copy(ei_hbm.at[1, pl.ds(start, CPT)], dstv)
    pltpu.sync_copy(ones_hbm, onesv)
    pltpu.sync_copy(zeros_hbm.at[pl.ds(s * RPT, RPT)],
                    deg_sh.at[pl.ds(s * RPT, RPT)])
    plsc.subcore_barrier()

    def body(g, carry):
        pltpu.sync_copy(onesv, deg_sh.at[dstv.at[g]], add=True)
        return carry

    lax.fori_loop(0, cnt, body, 0)
    plsc.subcore_barrier()
    pltpu.sync_copy(deg_sh.at[pl.ds(s * RPT, RPT)],
                    out_hbm.at[c, pl.ds(s * RPT, RPT)])


@functools.cache
def _deg_call():
    return pl.kernel(
        _deg_body,
        out_type=jax.ShapeDtypeStruct((NC, NP, 8), jnp.float32),
        mesh=plsc.VectorSubcoreMesh(
            core_axis_name="c", subcore_axis_name="s",
            num_cores=NC, num_subcores=NS),
        scratch_types=[
            pltpu.VMEM((CPT, CH), jnp.int32),
            pltpu.VMEM((CH, 8), jnp.float32),
            pltpu.VMEM_SHARED((NP, 8), jnp.float32),
        ],
        compiler_params=pltpu.CompilerParams(use_tc_tiling_on_sc=False),
    )


NBUF = 12


FULL = 72


def _edge_body(hp_hbm, ei_hbm, zeros_hbm, out_hbm,
               srcv, dstv, rows, gsems, ssems, agg_sh):
    c = lax.axis_index("c")
    s = lax.axis_index("s")
    wid = s * NC + c
    start, cnt = _tile_span(wid)
    d0 = pltpu.async_copy(ei_hbm.at[0, pl.ds(start, CPT)], srcv, gsems.at[0])
    d1 = pltpu.async_copy(ei_hbm.at[1, pl.ds(start, CPT)], dstv, gsems.at[1])
    d2 = pltpu.async_copy(zeros_hbm.at[pl.ds(s * RPT, RPT)],
                          agg_sh.at[pl.ds(s * RPT, RPT)], gsems.at[2])
    d0.wait()
    d1.wait()
    d2.wait()
    plsc.subcore_barrier()

    def gather(g, j):
        pltpu.async_copy(hp_hbm.at[srcv.at[g]], rows.at[j], gsems.at[j])

    def scatter(g, j):
        pltpu.async_copy(rows.at[j], agg_sh.at[dstv.at[g]], ssems.at[j],
                         add=True)

    def wait_gather(g, j):
        pltpu.make_async_copy(hp_hbm.at[srcv.at[g]], rows.at[j],
                              gsems.at[j]).wait()

    def wait_scatter(g, j):
        pltpu.make_async_copy(rows.at[j], agg_sh.at[dstv.at[g]],
                              ssems.at[j]).wait()

    for j in range(NBUF):
        gather(j, j)

    def body(p, carry):
        for j in range(NBUF):
            g = NBUF * p + j
            wait_gather(g, j)
            scatter(g, j)
        for j in range(NBUF):
            g = NBUF * p + j
            wait_scatter(g, j)
            nxt = g + NBUF

            @pl.when(nxt < cnt)
            def _():
                gather(nxt, j)
        return carry

    lax.fori_loop(0, FULL // NBUF, body, 0)
    for j in range(NBUF):
        g = FULL + j

        @pl.when(g < cnt)
        def _():
            wait_gather(g, j)
            scatter(g, j)
            wait_scatter(g, j)

    plsc.subcore_barrier()
    pltpu.sync_copy(agg_sh.at[pl.ds(s * RPT, RPT)],
                    out_hbm.at[c, pl.ds(s * RPT, RPT)])


@functools.cache
def _edge_call():
    return pl.kernel(
        _edge_body,
        out_type=jax.ShapeDtypeStruct((NC, NP, C), jnp.bfloat16),
        mesh=plsc.VectorSubcoreMesh(
            core_axis_name="c", subcore_axis_name="s",
            num_cores=NC, num_subcores=NS),
        scratch_types=[
            pltpu.VMEM((CPT, CH), jnp.int32),
            pltpu.VMEM((CPT, CH), jnp.int32),
            pltpu.VMEM((NBUF, CH, C), jnp.bfloat16),
            pltpu.SemaphoreType.DMA((NBUF,)),
            pltpu.SemaphoreType.DMA((NBUF,)),
            pltpu.VMEM_SHARED((NP, C), jnp.bfloat16),
        ],
        compiler_params=pltpu.CompilerParams(use_tc_tiling_on_sc=False),
    )


def _dinv_col(deg_ref):
    deg = deg_ref[0] + deg_ref[1]
    return lax.rsqrt(jnp.maximum(deg, 1.0))[:N, 0:1]


def _prep_tc(deg_ref, x_ref, w1_ref, w2_ref, b_ref, hp_ref, skip_ref):
    dcol = _dinv_col(deg_ref)
    x = x_ref[...]
    h = jnp.dot(x, w1_ref[...], preferred_element_type=jnp.float32)
    hp_ref[...] = (h * dcol).astype(jnp.bfloat16)
    skip_ref[...] = (
        jnp.dot(x, w2_ref[...], preferred_element_type=jnp.float32) + b_ref[...]
    ).astype(jnp.bfloat16)


_prep_call = pl.pallas_call(
    _prep_tc,
    out_shape=(
        jax.ShapeDtypeStruct((N, C), jnp.bfloat16),
        jax.ShapeDtypeStruct((N, C), jnp.bfloat16),
    ),
)


def _mid_tc(agg_ref, deg_ref, skip_ref, w1_ref, w2_ref, b_ref,
            hp_ref, skip2_ref):
    dcol = _dinv_col(deg_ref)
    agg = (agg_ref[0, :N].astype(jnp.float32)
           + agg_ref[1, :N].astype(jnp.float32))
    h = jnp.maximum(agg * dcol + skip_ref[...].astype(jnp.float32), 0.0)
    hp_ref[...] = (
        jnp.dot(h, w1_ref[...], preferred_element_type=jnp.float32) * dcol
    ).astype(jnp.bfloat16)
    skip2_ref[...] = (
        jnp.dot(h, w2_ref[...], preferred_element_type=jnp.float32) + b_ref[...]
    ).astype(jnp.bfloat16)


_mid_call = pl.pallas_call(
    _mid_tc,
    out_shape=(
        jax.ShapeDtypeStruct((N, C), jnp.bfloat16),
        jax.ShapeDtypeStruct((N, C), jnp.bfloat16),
    ),
)


def _final_tc(agg_ref, deg_ref, skip_ref, seg_ref, wd_ref, bd_ref, out_ref):
    dcol = _dinv_col(deg_ref)
    agg = (agg_ref[0, :N].astype(jnp.float32)
           + agg_ref[1, :N].astype(jnp.float32))
    h = jnp.maximum(agg * dcol + skip_ref[...].astype(jnp.float32), 0.0)
    gids = lax.broadcasted_iota(jnp.int32, (N, G), 1)
    m = (gids == seg_ref[...]).astype(jnp.float32)
    sums = lax.dot_general(m, h, (((0,), (0,)), ((), ())),
                           preferred_element_type=jnp.float32)
    counts = jnp.sum(m, axis=0)[:, None]
    pooled = sums / jnp.maximum(counts, 1.0)
    logits = (
        jnp.dot(pooled, wd_ref[...], preferred_element_type=jnp.float32)
        + bd_ref[...]
    )
    mx = jnp.max(logits, axis=1, keepdims=True)
    e = jnp.exp(logits - mx)
    out_ref[...] = e / jnp.sum(e, axis=1, keepdims=True)


_final_call = pl.pallas_call(
    _final_tc,
    out_shape=jax.ShapeDtypeStruct((G, L), jnp.float32),
)


def kernel(x, edge_index, i, W1_1, W2_1, b_1, W1_2, W2_2, b_2,
           W1_3, W2_3, b_3, Wd, bd):
    ei3 = edge_index.reshape(2, NCH, CH)
    ones8 = jnp.ones((CH, 8), jnp.float32)
    zeros8 = jnp.zeros((NP, 8), jnp.float32)
    zerosC = jnp.zeros((NP, C), jnp.bfloat16)

    deg_t = _deg_call()(ei3, ones8, zeros8)
    hp, skip = _prep_call(deg_t, x, W1_1, W2_1, b_1.reshape(1, C))
    agg = _edge_call()(hp, ei3, zerosC)
    hp, skip = _mid_call(agg, deg_t, skip, W1_2, W2_2, b_2.reshape(1, C))
    agg = _edge_call()(hp, ei3, zerosC)
    hp, skip = _mid_call(agg, deg_t, skip, W1_3, W2_3, b_3.reshape(1, C))
    agg = _edge_call()(hp, ei3, zerosC)
    return _final_call(agg, deg_t, skip, i.reshape(N, 1), Wd,
                       bd.reshape(1, L))
